# Initial kernel scaffold; baseline (speedup 1.0000x reference)
import jax
import jax.numpy as jnp
from jax.experimental import pallas as pl


def kernel(t, embed_table):
    raise NotImplementedError("write your pallas kernel here")


# speedup vs baseline: 1.9648x; 1.9648x over previous
import functools

import numpy as np
import jax
import jax.numpy as jnp
from jax import lax
from jax.experimental import pallas as pl
from jax.experimental.pallas import tpu as pltpu
from jax.experimental.pallas import tpu_sc as plsc

MAXF = 1024
D = 256
NF = 10
DOUT = D + 2 * NF
DPAD = 384
B = 16384
F = 3
B2 = B * (F - 1)

NC = 2
NS = 16
L = 16
NW = NC * NS

ROWS_W = B2 // NW
BPW = B // NW
CH = 128
NCH = ROWS_W // CH
GPC = CH // L


def _fourier_lut() -> np.ndarray:
    time_resolution = 2.0 ** np.ceil(np.log2(MAXF))
    powers = 2.0 ** np.arange(NF)
    coefs = (powers * np.pi / time_resolution).astype(np.float32)
    d = np.arange(MAXF, dtype=np.float32)
    raw = (d[:, None] * coefs[None, :]).astype(np.float32)
    return np.concatenate(
        [np.sin(raw.astype(np.float64)), np.cos(raw.astype(np.float64))],
        axis=1).astype(np.float32)


_LUT = _fourier_lut()


@functools.partial(
    pl.kernel,
    out_type=jax.ShapeDtypeStruct((B2, DPAD), jnp.float32),
    mesh=plsc.VectorSubcoreMesh(core_axis_name="c", subcore_axis_name="s",
                                num_cores=NC, num_subcores=NS),
    scratch_types=[
        pltpu.VMEM((BPW * F,), jnp.int32),
        pltpu.VMEM((NCH, CH), jnp.int32),
        pltpu.VMEM((2, CH, DPAD), jnp.float32),
        pltpu.SemaphoreType.DMA,
        pltpu.SemaphoreType.DMA,
        pltpu.SemaphoreType.DMA,
        pltpu.SemaphoreType.DMA,
    ],
    compiler_params=pltpu.CompilerParams(needs_layout_passes=False),
)
def _sc_lookup(t_hbm, tab_hbm, out_hbm, t_v, idx_v, buf_v,
               gsem0, gsem1, osem0, osem1):
    wid = lax.axis_index("s") * NC + lax.axis_index("c")
    pltpu.sync_copy(t_hbm.at[pl.ds(wid * (BPW * F), BPW * F)], t_v)

    lanes = jnp.arange(L, dtype=jnp.int32)
    for c in range(NCH):
        for g in range(GPC):
            r = lanes + (c * CH + g * L)
            pos = (r >> 1) * 3 + (r & 1)
            lo = plsc.load_gather(t_v, [pos])
            hi = plsc.load_gather(t_v, [pos + 1])
            idx_v[c, pl.ds(g * L, L)] = hi - lo

    gsems = (gsem0, gsem1)
    osems = (osem0, osem1)
    obase = wid * ROWS_W
    gd = [None, None]
    od = [None, None]
    gd[0] = pltpu.async_copy(tab_hbm.at[idx_v.at[0]], buf_v.at[0], gsems[0])
    for c in range(NCH):
        s = c % 2
        ns = (c + 1) % 2
        if c + 1 < NCH:
            if od[ns] is not None:
                od[ns].wait()
            gd[ns] = pltpu.async_copy(
                tab_hbm.at[idx_v.at[c + 1]], buf_v.at[ns], gsems[ns])
        gd[s].wait()
        od[s] = pltpu.async_copy(
            buf_v.at[s], out_hbm.at[pl.ds(obase + c * CH, CH)], osems[s])
    od[0].wait()
    od[1].wait()


def kernel(t, embed_table):
    fused = jnp.concatenate(
        [embed_table, jnp.asarray(_LUT, dtype=jnp.float32),
         jnp.zeros((MAXF, DPAD - DOUT), dtype=jnp.float32)], axis=1)
    out = _sc_lookup(t.astype(jnp.int32).reshape(-1), fused)
    return out[:, :DOUT].reshape(B, -1)


# speedup vs baseline: 2.7070x; 1.3778x over previous
import functools

import numpy as np
import jax
import jax.numpy as jnp
from jax import lax
from jax.experimental import pallas as pl
from jax.experimental.pallas import tpu as pltpu
from jax.experimental.pallas import tpu_sc as plsc

MAXF = 1024
D = 256
NF = 10
DOUT = D + 2 * NF
DPAD = 384
B = 16384
F = 3
DW = 2 * DOUT

NC = 2
NS = 16
L = 16
NW = NC * NS

BPW = B // NW
RC = 32
NCH = BPW // RC
NFULL = DOUT // L
REM = DOUT - NFULL * L


def _fourier_lut() -> np.ndarray:
    time_resolution = 2.0 ** np.ceil(np.log2(MAXF))
    powers = 2.0 ** np.arange(NF)
    coefs = (powers * np.pi / time_resolution).astype(np.float32)
    d = np.arange(MAXF, dtype=np.float32)
    raw = (d[:, None] * coefs[None, :]).astype(np.float32)
    lut = np.zeros((MAXF, DPAD - D), dtype=np.float32)
    lut[:, :NF] = np.sin(raw.astype(np.float64)).astype(np.float32)
    lut[:, NF:2 * NF] = np.cos(raw.astype(np.float64)).astype(np.float32)
    return lut


_LUT = _fourier_lut()


@functools.partial(
    pl.kernel,
    out_type=jax.ShapeDtypeStruct((B, DW), jnp.float32),
    mesh=plsc.VectorSubcoreMesh(core_axis_name="c", subcore_axis_name="s",
                                num_cores=NC, num_subcores=NS),
    scratch_types=[
        pltpu.VMEM((BPW * F,), jnp.int32),
        pltpu.VMEM((NCH, RC), jnp.int32),
        pltpu.VMEM((NCH, RC), jnp.int32),
        pltpu.VMEM((3, RC, DW), jnp.float32),
        pltpu.VMEM((3, RC, DPAD), jnp.float32),
        pltpu.SemaphoreType.DMA,
        pltpu.SemaphoreType.DMA,
        pltpu.SemaphoreType.DMA,
        pltpu.SemaphoreType.DMA,
        pltpu.SemaphoreType.DMA,
        pltpu.SemaphoreType.DMA,
        pltpu.SemaphoreType.DMA,
        pltpu.SemaphoreType.DMA,
        pltpu.SemaphoreType.DMA,
    ],
    compiler_params=pltpu.CompilerParams(needs_layout_passes=False),
)
def _sc_encode(t_hbm, tab_hbm, out_hbm, t_v, idxe_v, idxo_v, abuf_v, obuf_v,
               ge0, ge1, ge2, go0, go1, go2, os0, os1, os2):
    wid = lax.axis_index("s") * NC + lax.axis_index("c")
    pltpu.sync_copy(t_hbm.at[pl.ds(wid * (BPW * F), BPW * F)], t_v)

    lanes = jnp.arange(L, dtype=jnp.int32)
    for g in range(BPW // L):
        pos = (lanes + g * L) * 3
        t0 = plsc.load_gather(t_v, [pos])
        t1 = plsc.load_gather(t_v, [pos + 1])
        t2 = plsc.load_gather(t_v, [pos + 2])
        c, col = divmod(g * L, RC)
        idxe_v[c, pl.ds(col, L)] = t1 - t0
        idxo_v[c, pl.ds(col, L)] = t2 - t1

    gsems = ((ge0, go0), (ge1, go1), (ge2, go2))
    osems = (os0, os1, os2)

    def start_gathers(c, s):
        ge = pltpu.async_copy(tab_hbm.at[idxe_v.at[c]],
                              abuf_v.at[s, :, pl.ds(0, DPAD)], gsems[s][0])
        go = pltpu.async_copy(tab_hbm.at[idxo_v.at[c]],
                              obuf_v.at[s], gsems[s][1])
        return ge, go

    cross = {g for g in range(NFULL) if (DOUT + L * g) % 128 > 128 - L}
    rem_m = lanes < REM
    rem_src = (NFULL * L) + jnp.where(rem_m, lanes, 0)
    rem_dst = rem_src + DOUT

    def assemble(s):
        def row(i, _):
            rows16 = jnp.full((L,), i, jnp.int32)
            for g in range(NFULL):
                x = obuf_v[s, i, pl.ds(L * g, L)]
                if g in cross:
                    plsc.store_scatter(
                        abuf_v.at[s], [rows16, lanes + (DOUT + L * g)], x)
                else:
                    abuf_v[s, i, pl.ds(DOUT + L * g, L)] = x
            x = plsc.load_gather(obuf_v.at[s], [rows16, rem_src], mask=rem_m)
            plsc.store_scatter(abuf_v.at[s], [rows16, rem_dst], x, mask=rem_m)
            return ()
        lax.fori_loop(0, RC, row, (), unroll=1)

    obase = wid * BPW
    gd = [None, None, None]
    od = [None, None, None]
    gd[0] = start_gathers(0, 0)
    gd[1] = start_gathers(1, 1)
    for c in range(NCH):
        s = c % 3
        gd[s][0].wait()
        gd[s][1].wait()
        assemble(s)
        od[s] = pltpu.async_copy(
            abuf_v.at[s], out_hbm.at[pl.ds(obase + c * RC, RC)], osems[s])
        if c + 2 < NCH:
            ns = (c + 2) % 3
            if od[ns] is not None:
                od[ns].wait()
            gd[ns] = start_gathers(c + 2, ns)
    od[(NCH - 2) % 3].wait()
    od[(NCH - 1) % 3].wait()


def kernel(t, embed_table):
    fused = jnp.concatenate(
        [embed_table, jnp.asarray(_LUT, dtype=jnp.float32)], axis=1)
    return _sc_encode(t.astype(jnp.int32).reshape(-1), fused)
